# baseline (speedup 1.0000x reference)
import jax
import jax.numpy as jnp
from jax.experimental import pallas as pl
from jax.experimental.pallas import tpu as pltpu

_HI = jax.lax.Precision.HIGHEST


def _gelu(v):
    return 0.5 * v * (1.0 + jax.lax.erf(v * 0.7071067811865475))


def _router_kernel(x_ref, emb_ref, sel_ref, *, bsz, ntok, dim, nexp):
    T = bsz * ntok
    x2 = x_ref[:]
    logits = jnp.dot(x2.astype(jnp.bfloat16), emb_ref[:].astype(jnp.bfloat16).T,
                     preferred_element_type=jnp.float32)
    col = jax.lax.broadcasted_iota(jnp.int32, (T, nexp), 1)
    m1 = jnp.max(logits, axis=1, keepdims=True)
    i1 = jnp.min(jnp.where(logits == m1, col, nexp), axis=1, keepdims=True)
    masked = jnp.where(col == i1, -jnp.inf, logits)
    m2 = jnp.max(masked, axis=1, keepdims=True)
    i2 = jnp.min(jnp.where(masked == m2, col, nexp), axis=1, keepdims=True)
    t = jax.lax.broadcasted_iota(jnp.int32, (T, 1), 0)
    base = t - t % ntok
    src = jnp.concatenate([(base + i1).astype(jnp.float32),
                           (base + i2).astype(jnp.float32)], axis=1)
    q = (t % bsz) * ntok + t // bsz
    colT = jax.lax.broadcasted_iota(jnp.int32, (T, T), 1)
    perm = (colT == q).astype(jnp.float32)
    srcp = jnp.dot(perm, src, preferred_element_type=jnp.float32,
                   precision=_HI)
    s1 = srcp[:, 0:1].astype(jnp.int32)
    s2 = srcp[:, 1:2].astype(jnp.int32)
    oh1 = (colT == s1).astype(jnp.float32)
    oh2 = (colT == s2).astype(jnp.float32)
    g1 = jnp.dot(oh1, x2, preferred_element_type=jnp.float32, precision=_HI)
    g2 = jnp.dot(oh2, x2, preferred_element_type=jnp.float32, precision=_HI)
    sel = jnp.concatenate([g1, g2], axis=1)
    sel_ref[:] = sel.reshape(nexp, bsz, 2 * dim)


def _sw_kernel(x_ref, w1a_ref, w1b_ref, w1c_ref, w1d_ref,
               b1a_ref, b1b_ref, b1c_ref, b1d_ref,
               w2a_ref, w2b_ref, w2c_ref, w2d_ref,
               b2_ref, wts_ref, acc_ref):
    s = pl.program_id(0)
    xb = x_ref[:].astype(jnp.bfloat16)
    contrib = 0.0
    for w1_ref, b1_ref, w2_ref in ((w1a_ref, b1a_ref, w2a_ref),
                                   (w1b_ref, b1b_ref, w2b_ref),
                                   (w1c_ref, b1c_ref, w2c_ref),
                                   (w1d_ref, b1d_ref, w2d_ref)):
        h = _gelu(jnp.dot(xb, w1_ref[:].astype(jnp.bfloat16).T,
                          preferred_element_type=jnp.float32) + b1_ref[:])
        contrib = contrib + jnp.dot(
            h.astype(jnp.bfloat16), w2_ref[:].astype(jnp.bfloat16).T,
            preferred_element_type=jnp.float32)

    @pl.when(s == 0)
    def _():
        acc_ref[:] = contrib

    @pl.when(s > 0)
    def _():
        acc_ref[:] = acc_ref[:] + contrib

    @pl.when(s == pl.num_programs(0) - 1)
    def _():
        logits = acc_ref[:] + b2_ref[:]
        m = jnp.max(logits, axis=1, keepdims=True)
        ez = jnp.exp(logits - m)
        wts_ref[:] = ez / jnp.sum(ez, axis=1, keepdims=True)


def _fc1_kernel(sel_ref, w_ref, b_ref, h_ref):
    sb = sel_ref[0].astype(jnp.bfloat16)
    wb = w_ref[0].astype(jnp.bfloat16)
    h = jnp.dot(sb, wb.T, preferred_element_type=jnp.float32) + b_ref[0]
    h_ref[0] = _gelu(h)


def _fc2_kernel(h_ref, w_ref, b_ref, wts_ref, out_ref, *, nexp):
    e = pl.program_id(0)
    hb = h_ref[0].astype(jnp.bfloat16)
    wb = w_ref[0].astype(jnp.bfloat16)
    r = jnp.dot(hb, wb.T, preferred_element_type=jnp.float32) + b_ref[0]
    ecol = jax.lax.broadcasted_iota(jnp.int32, (nexp, 1), 0)
    onehot = (ecol == e).astype(jnp.float32)
    wcol = jnp.dot(wts_ref[:], onehot, preferred_element_type=jnp.float32,
                   precision=_HI)
    contrib = r * wcol

    @pl.when(e == 0)
    def _():
        out_ref[:] = contrib

    @pl.when(e > 0)
    def _():
        out_ref[:] = out_ref[:] + contrib


def _head_kernel(ws_ref, w1_ref, b1_ref, w2_ref, b2_ref, out_ref):
    wsb = ws_ref[:].astype(jnp.bfloat16)
    h = jnp.dot(wsb, w1_ref[:].astype(jnp.bfloat16).T,
                preferred_element_type=jnp.float32) + b1_ref[:]
    hb = _gelu(h).astype(jnp.bfloat16)
    out_ref[:] = jnp.dot(hb, w2_ref[:].astype(jnp.bfloat16).T,
                         preferred_element_type=jnp.float32) + b2_ref[:]


def kernel(x, expert_emb, exp_fc1_w, exp_fc1_b, exp_fc2_w, exp_fc2_b,
           sw_fc1_w, sw_fc1_b, sw_fc2_w, sw_fc2_b,
           ch_fc1_w, ch_fc1_b, ch_fc2_w, ch_fc2_b):
    import functools
    bsz, ntok, dim = x.shape
    nexp = expert_emb.shape[0]
    ed = exp_fc1_w.shape[1]
    ncls = ch_fc2_w.shape[0]
    f32 = jnp.float32

    x_tok = x.reshape(bsz * ntok, dim)
    x_flat = x.reshape(bsz, ntok * dim)

    sel = pl.pallas_call(
        functools.partial(_router_kernel, bsz=bsz, ntok=ntok, dim=dim,
                          nexp=nexp),
        out_shape=jax.ShapeDtypeStruct((nexp, bsz, ed), f32),
    )(x_tok, expert_emb)

    SWB = 128
    nsteps = (ntok * dim) // (4 * SWB)
    b1_2d = sw_fc1_b.reshape(1, -1)

    def _w1spec(j):
        return pl.BlockSpec((SWB, ntok * dim), lambda s: (s + j * 16, 0))

    def _b1spec(j):
        return pl.BlockSpec((1, SWB), lambda s: (0, s + j * 16))

    def _w2spec(j):
        return pl.BlockSpec((nexp, SWB), lambda s: (0, s + j * 16))

    wts = pl.pallas_call(
        _sw_kernel,
        grid=(nsteps,),
        in_specs=[
            pl.BlockSpec((bsz, ntok * dim), lambda s: (0, 0)),
            _w1spec(0), _w1spec(1), _w1spec(2), _w1spec(3),
            _b1spec(0), _b1spec(1), _b1spec(2), _b1spec(3),
            _w2spec(0), _w2spec(1), _w2spec(2), _w2spec(3),
            pl.BlockSpec((1, nexp), lambda s: (0, 0)),
        ],
        out_specs=pl.BlockSpec((bsz, nexp), lambda s: (0, 0)),
        out_shape=jax.ShapeDtypeStruct((bsz, nexp), f32),
        scratch_shapes=[pltpu.VMEM((bsz, nexp), f32)],
        compiler_params=pltpu.CompilerParams(
            vmem_limit_bytes=60 * 1024 * 1024),
    )(x_flat, sw_fc1_w, sw_fc1_w, sw_fc1_w, sw_fc1_w,
      b1_2d, b1_2d, b1_2d, b1_2d,
      sw_fc2_w, sw_fc2_w, sw_fc2_w, sw_fc2_w,
      sw_fc2_b.reshape(1, -1))

    h1 = pl.pallas_call(
        _fc1_kernel,
        grid=(nexp,),
        in_specs=[
            pl.BlockSpec((1, bsz, ed), lambda e: (e, 0, 0)),
            pl.BlockSpec((1, ed, ed), lambda e: (e, 0, 0)),
            pl.BlockSpec((1, 1, ed), lambda e: (e, 0, 0)),
        ],
        out_specs=pl.BlockSpec((1, bsz, ed), lambda e: (e, 0, 0)),
        out_shape=jax.ShapeDtypeStruct((nexp, bsz, ed), f32),
        compiler_params=pltpu.CompilerParams(
            vmem_limit_bytes=60 * 1024 * 1024),
    )(sel, exp_fc1_w, exp_fc1_b.reshape(nexp, 1, ed))

    ws = pl.pallas_call(
        functools.partial(_fc2_kernel, nexp=nexp),
        grid=(nexp,),
        in_specs=[
            pl.BlockSpec((1, bsz, ed), lambda e: (e, 0, 0)),
            pl.BlockSpec((1, ed, ed), lambda e: (e, 0, 0)),
            pl.BlockSpec((1, 1, ed), lambda e: (e, 0, 0)),
            pl.BlockSpec((bsz, nexp), lambda e: (0, 0)),
        ],
        out_specs=pl.BlockSpec((bsz, ed), lambda e: (0, 0)),
        out_shape=jax.ShapeDtypeStruct((bsz, ed), f32),
        compiler_params=pltpu.CompilerParams(
            vmem_limit_bytes=60 * 1024 * 1024),
    )(h1, exp_fc2_w, exp_fc2_b.reshape(nexp, 1, ed), wts)

    out = pl.pallas_call(
        _head_kernel,
        in_specs=[
            pl.BlockSpec((bsz, ed), lambda: (0, 0)),
            pl.BlockSpec((ed, ed), lambda: (0, 0)),
            pl.BlockSpec((1, ed), lambda: (0, 0)),
            pl.BlockSpec((ncls, ed), lambda: (0, 0)),
            pl.BlockSpec((1, ncls), lambda: (0, 0)),
        ],
        out_specs=pl.BlockSpec((bsz, ncls), lambda: (0, 0)),
        out_shape=jax.ShapeDtypeStruct((bsz, ncls), f32),
        compiler_params=pltpu.CompilerParams(
            vmem_limit_bytes=60 * 1024 * 1024),
    )(ws, ch_fc1_w, ch_fc1_b.reshape(1, -1), ch_fc2_w,
      ch_fc2_b.reshape(1, -1))
    return out


# speedup vs baseline: 1.0448x; 1.0448x over previous
import jax
import jax.numpy as jnp
from jax.experimental import pallas as pl
from jax.experimental.pallas import tpu as pltpu

_HI = jax.lax.Precision.HIGHEST


def _gelu(v):
    return 0.5 * v * (1.0 + jax.lax.erf(v * 0.7071067811865475))


def _router_kernel(x_ref, emb_ref, sel_ref, *, bsz, ntok, dim, nexp):
    T = bsz * ntok
    x2 = x_ref[:]
    logits = jnp.dot(x2.astype(jnp.bfloat16), emb_ref[:].astype(jnp.bfloat16).T,
                     preferred_element_type=jnp.float32)
    col = jax.lax.broadcasted_iota(jnp.int32, (T, nexp), 1)
    m1 = jnp.max(logits, axis=1, keepdims=True)
    i1 = jnp.min(jnp.where(logits == m1, col, nexp), axis=1, keepdims=True)
    masked = jnp.where(col == i1, -jnp.inf, logits)
    m2 = jnp.max(masked, axis=1, keepdims=True)
    i2 = jnp.min(jnp.where(masked == m2, col, nexp), axis=1, keepdims=True)
    t = jax.lax.broadcasted_iota(jnp.int32, (T, 1), 0)
    base = t - t % ntok
    src = jnp.concatenate([(base + i1).astype(jnp.float32),
                           (base + i2).astype(jnp.float32)], axis=1)
    q = (t % bsz) * ntok + t // bsz
    colT = jax.lax.broadcasted_iota(jnp.int32, (T, T), 1)
    perm = (colT == q).astype(jnp.float32)
    srcp = jnp.dot(perm, src, preferred_element_type=jnp.float32,
                   precision=_HI)
    s1 = srcp[:, 0:1].astype(jnp.int32)
    s2 = srcp[:, 1:2].astype(jnp.int32)
    oh1 = (colT == s1).astype(jnp.float32)
    oh2 = (colT == s2).astype(jnp.float32)
    g1 = jnp.dot(oh1, x2, preferred_element_type=jnp.float32, precision=_HI)
    g2 = jnp.dot(oh2, x2, preferred_element_type=jnp.float32, precision=_HI)
    sel = jnp.concatenate([g1, g2], axis=1)
    sel_ref[:] = sel.reshape(nexp, bsz, 2 * dim)


def _sw_kernel(x_ref, w1a_ref, w1b_ref, b1a_ref, b1b_ref,
               w2a_ref, w2b_ref, b2_ref, wts_ref, acc_ref):
    s = pl.program_id(0)
    xb = x_ref[:].astype(jnp.bfloat16)
    contrib = 0.0
    for w1_ref, b1_ref, w2_ref in ((w1a_ref, b1a_ref, w2a_ref),
                                   (w1b_ref, b1b_ref, w2b_ref)):
        h = _gelu(jnp.dot(xb, w1_ref[:].astype(jnp.bfloat16).T,
                          preferred_element_type=jnp.float32) + b1_ref[:])
        contrib = contrib + jnp.dot(
            h.astype(jnp.bfloat16), w2_ref[:].astype(jnp.bfloat16).T,
            preferred_element_type=jnp.float32)

    @pl.when(s == 0)
    def _():
        acc_ref[:] = contrib

    @pl.when(s > 0)
    def _():
        acc_ref[:] = acc_ref[:] + contrib

    @pl.when(s == pl.num_programs(0) - 1)
    def _():
        logits = acc_ref[:] + b2_ref[:]
        m = jnp.max(logits, axis=1, keepdims=True)
        ez = jnp.exp(logits - m)
        wts_ref[:] = ez / jnp.sum(ez, axis=1, keepdims=True)


def _fc1_kernel(sel_ref, wt_ref, wb_ref, b_ref, h_ref):
    sb = sel_ref[0].astype(jnp.bfloat16)
    ht = jnp.dot(sb, wt_ref[0].astype(jnp.bfloat16).T,
                 preferred_element_type=jnp.float32)
    hb = jnp.dot(sb, wb_ref[0].astype(jnp.bfloat16).T,
                 preferred_element_type=jnp.float32)
    h = jnp.concatenate([ht, hb], axis=1) + b_ref[0]
    h_ref[0] = _gelu(h)


def _fc2_kernel(h_ref, wt_ref, wb_ref, b_ref, wts_ref, out_ref, *, nexp):
    e = pl.program_id(0)
    hb16 = h_ref[0].astype(jnp.bfloat16)
    rt = jnp.dot(hb16, wt_ref[0].astype(jnp.bfloat16).T,
                 preferred_element_type=jnp.float32)
    rb = jnp.dot(hb16, wb_ref[0].astype(jnp.bfloat16).T,
                 preferred_element_type=jnp.float32)
    r = jnp.concatenate([rt, rb], axis=1) + b_ref[0]
    ecol = jax.lax.broadcasted_iota(jnp.int32, (nexp, 1), 0)
    onehot = (ecol == e).astype(jnp.float32)
    wcol = jnp.dot(wts_ref[:], onehot, preferred_element_type=jnp.float32,
                   precision=_HI)
    contrib = r * wcol

    @pl.when(e == 0)
    def _():
        out_ref[:] = contrib

    @pl.when(e > 0)
    def _():
        out_ref[:] = out_ref[:] + contrib


def _head_kernel(ws_ref, w1_ref, b1_ref, w2_ref, b2_ref, out_ref):
    wsb = ws_ref[:].astype(jnp.bfloat16)
    h = jnp.dot(wsb, w1_ref[:].astype(jnp.bfloat16).T,
                preferred_element_type=jnp.float32) + b1_ref[:]
    hb = _gelu(h).astype(jnp.bfloat16)
    out_ref[:] = jnp.dot(hb, w2_ref[:].astype(jnp.bfloat16).T,
                         preferred_element_type=jnp.float32) + b2_ref[:]


def kernel(x, expert_emb, exp_fc1_w, exp_fc1_b, exp_fc2_w, exp_fc2_b,
           sw_fc1_w, sw_fc1_b, sw_fc2_w, sw_fc2_b,
           ch_fc1_w, ch_fc1_b, ch_fc2_w, ch_fc2_b):
    import functools
    bsz, ntok, dim = x.shape
    nexp = expert_emb.shape[0]
    ed = exp_fc1_w.shape[1]
    ncls = ch_fc2_w.shape[0]
    f32 = jnp.float32

    x_tok = x.reshape(bsz * ntok, dim)
    x_flat = x.reshape(bsz, ntok * dim)

    sel = pl.pallas_call(
        functools.partial(_router_kernel, bsz=bsz, ntok=ntok, dim=dim,
                          nexp=nexp),
        out_shape=jax.ShapeDtypeStruct((nexp, bsz, ed), f32),
    )(x_tok, expert_emb)

    SWB = 256
    nsteps = (ntok * dim) // (2 * SWB)
    b1_2d = sw_fc1_b.reshape(1, -1)

    def _w1spec(j):
        return pl.BlockSpec((SWB, ntok * dim), lambda s: (s + j * 16, 0))

    def _b1spec(j):
        return pl.BlockSpec((1, SWB), lambda s: (0, s + j * 16))

    def _w2spec(j):
        return pl.BlockSpec((nexp, SWB), lambda s: (0, s + j * 16))

    wts = pl.pallas_call(
        _sw_kernel,
        grid=(nsteps,),
        in_specs=[
            pl.BlockSpec((bsz, ntok * dim), lambda s: (0, 0)),
            _w1spec(0), _w1spec(1),
            _b1spec(0), _b1spec(1),
            _w2spec(0), _w2spec(1),
            pl.BlockSpec((1, nexp), lambda s: (0, 0)),
        ],
        out_specs=pl.BlockSpec((bsz, nexp), lambda s: (0, 0)),
        out_shape=jax.ShapeDtypeStruct((bsz, nexp), f32),
        scratch_shapes=[pltpu.VMEM((bsz, nexp), f32)],
        compiler_params=pltpu.CompilerParams(
            vmem_limit_bytes=60 * 1024 * 1024),
    )(x_flat, sw_fc1_w, sw_fc1_w, b1_2d, b1_2d,
      sw_fc2_w, sw_fc2_w, sw_fc2_b.reshape(1, -1))

    hed = ed // 2
    h1 = pl.pallas_call(
        _fc1_kernel,
        grid=(nexp,),
        in_specs=[
            pl.BlockSpec((1, bsz, ed), lambda e: (e, 0, 0)),
            pl.BlockSpec((1, hed, ed), lambda e: (e, 0, 0)),
            pl.BlockSpec((1, hed, ed), lambda e: (e, 1, 0)),
            pl.BlockSpec((1, 1, ed), lambda e: (e, 0, 0)),
        ],
        out_specs=pl.BlockSpec((1, bsz, ed), lambda e: (e, 0, 0)),
        out_shape=jax.ShapeDtypeStruct((nexp, bsz, ed), f32),
        compiler_params=pltpu.CompilerParams(
            vmem_limit_bytes=60 * 1024 * 1024),
    )(sel, exp_fc1_w, exp_fc1_w, exp_fc1_b.reshape(nexp, 1, ed))

    ws = pl.pallas_call(
        functools.partial(_fc2_kernel, nexp=nexp),
        grid=(nexp,),
        in_specs=[
            pl.BlockSpec((1, bsz, ed), lambda e: (e, 0, 0)),
            pl.BlockSpec((1, hed, ed), lambda e: (e, 0, 0)),
            pl.BlockSpec((1, hed, ed), lambda e: (e, 1, 0)),
            pl.BlockSpec((1, 1, ed), lambda e: (e, 0, 0)),
            pl.BlockSpec((bsz, nexp), lambda e: (0, 0)),
        ],
        out_specs=pl.BlockSpec((bsz, ed), lambda e: (0, 0)),
        out_shape=jax.ShapeDtypeStruct((bsz, ed), f32),
        compiler_params=pltpu.CompilerParams(
            vmem_limit_bytes=60 * 1024 * 1024),
    )(h1, exp_fc2_w, exp_fc2_w, exp_fc2_b.reshape(nexp, 1, ed), wts)

    out = pl.pallas_call(
        _head_kernel,
        in_specs=[
            pl.BlockSpec((bsz, ed), lambda: (0, 0)),
            pl.BlockSpec((ed, ed), lambda: (0, 0)),
            pl.BlockSpec((1, ed), lambda: (0, 0)),
            pl.BlockSpec((ncls, ed), lambda: (0, 0)),
            pl.BlockSpec((1, ncls), lambda: (0, 0)),
        ],
        out_specs=pl.BlockSpec((bsz, ncls), lambda: (0, 0)),
        out_shape=jax.ShapeDtypeStruct((bsz, ncls), f32),
        compiler_params=pltpu.CompilerParams(
            vmem_limit_bytes=60 * 1024 * 1024),
    )(ws, ch_fc1_w, ch_fc1_b.reshape(1, -1), ch_fc2_w,
      ch_fc2_b.reshape(1, -1))
    return out


# speedup vs baseline: 1.0509x; 1.0059x over previous
import jax
import jax.numpy as jnp
from jax.experimental import pallas as pl
from jax.experimental.pallas import tpu as pltpu

_HI = jax.lax.Precision.HIGHEST


def _gelu(v):
    return 0.5 * v * (1.0 + jax.lax.erf(v * 0.7071067811865475))


def _router_kernel(x_ref, emb_ref, sel_ref, *, bsz, ntok, dim, nexp):
    T = bsz * ntok
    x2 = x_ref[:]
    logits = jnp.dot(x2.astype(jnp.bfloat16), emb_ref[:].astype(jnp.bfloat16).T,
                     preferred_element_type=jnp.float32)
    col = jax.lax.broadcasted_iota(jnp.int32, (T, nexp), 1)
    m1 = jnp.max(logits, axis=1, keepdims=True)
    i1 = jnp.min(jnp.where(logits == m1, col, nexp), axis=1, keepdims=True)
    masked = jnp.where(col == i1, -jnp.inf, logits)
    m2 = jnp.max(masked, axis=1, keepdims=True)
    i2 = jnp.min(jnp.where(masked == m2, col, nexp), axis=1, keepdims=True)
    t = jax.lax.broadcasted_iota(jnp.int32, (T, 1), 0)
    base = t - t % ntok
    src = jnp.concatenate([(base + i1).astype(jnp.float32),
                           (base + i2).astype(jnp.float32)], axis=1)
    q = (t % bsz) * ntok + t // bsz
    colT = jax.lax.broadcasted_iota(jnp.int32, (T, T), 1)
    perm = (colT == q).astype(jnp.float32)
    srcp = jnp.dot(perm, src, preferred_element_type=jnp.float32,
                   precision=_HI)
    s1 = srcp[:, 0:1].astype(jnp.int32)
    s2 = srcp[:, 1:2].astype(jnp.int32)
    oh1 = (colT == s1).astype(jnp.float32)
    oh2 = (colT == s2).astype(jnp.float32)
    g1 = jnp.dot(oh1, x2, preferred_element_type=jnp.float32, precision=_HI)
    g2 = jnp.dot(oh2, x2, preferred_element_type=jnp.float32, precision=_HI)
    sel = jnp.concatenate([g1, g2], axis=1)
    sel_ref[:] = sel.reshape(nexp, bsz, 2 * dim)


def _sw_kernel(x_ref, w1a_ref, w1b_ref, b1a_ref, b1b_ref,
               w2a_ref, w2b_ref, b2_ref, wts_ref, acc_ref):
    s = pl.program_id(0)
    xb = x_ref[:].astype(jnp.bfloat16)
    contrib = 0.0
    for w1_ref, b1_ref, w2_ref in ((w1a_ref, b1a_ref, w2a_ref),
                                   (w1b_ref, b1b_ref, w2b_ref)):
        h = _gelu(jnp.dot(xb, w1_ref[:].astype(jnp.bfloat16).T,
                          preferred_element_type=jnp.float32) + b1_ref[:])
        contrib = contrib + jnp.dot(
            h.astype(jnp.bfloat16), w2_ref[:].astype(jnp.bfloat16).T,
            preferred_element_type=jnp.float32)

    @pl.when(s == 0)
    def _():
        acc_ref[:] = contrib

    @pl.when(s > 0)
    def _():
        acc_ref[:] = acc_ref[:] + contrib

    @pl.when(s == pl.num_programs(0) - 1)
    def _():
        logits = acc_ref[:] + b2_ref[:]
        m = jnp.max(logits, axis=1, keepdims=True)
        ez = jnp.exp(logits - m)
        wts_ref[:] = ez / jnp.sum(ez, axis=1, keepdims=True)


def _fc1_kernel(sel_ref, wt_ref, wb_ref, b_ref, h_ref):
    sb = sel_ref[0].astype(jnp.bfloat16)
    ht = jnp.dot(sb, wt_ref[0].astype(jnp.bfloat16).T,
                 preferred_element_type=jnp.float32)
    hb = jnp.dot(sb, wb_ref[0].astype(jnp.bfloat16).T,
                 preferred_element_type=jnp.float32)
    h = jnp.concatenate([ht, hb], axis=1) + b_ref[0]
    h_ref[0] = _gelu(h)


def _fc2_kernel(h_ref, wt_ref, wb_ref, b_ref, wts_ref, out_ref, *, nexp):
    e = pl.program_id(0)
    hb16 = h_ref[0].astype(jnp.bfloat16)
    rt = jnp.dot(hb16, wt_ref[0].astype(jnp.bfloat16).T,
                 preferred_element_type=jnp.float32)
    rb = jnp.dot(hb16, wb_ref[0].astype(jnp.bfloat16).T,
                 preferred_element_type=jnp.float32)
    r = jnp.concatenate([rt, rb], axis=1) + b_ref[0]
    ecol = jax.lax.broadcasted_iota(jnp.int32, (nexp, 1), 0)
    onehot = (ecol == e).astype(jnp.float32)
    wcol = jnp.dot(wts_ref[:], onehot, preferred_element_type=jnp.float32,
                   precision=_HI)
    contrib = r * wcol

    @pl.when(e == 0)
    def _():
        out_ref[:] = contrib

    @pl.when(e > 0)
    def _():
        out_ref[:] = out_ref[:] + contrib


def _head_kernel(ws_ref, c1a_ref, c1b_ref, b1_ref, c2_ref, b2_ref, out_ref,
                 hid_ref):
    s = pl.program_id(0)

    @pl.when(s < 2)
    def _():
        wsb = ws_ref[:].astype(jnp.bfloat16)
        ht = jnp.dot(wsb, c1a_ref[:].astype(jnp.bfloat16).T,
                     preferred_element_type=jnp.float32)
        hb = jnp.dot(wsb, c1b_ref[:].astype(jnp.bfloat16).T,
                     preferred_element_type=jnp.float32)
        hid_ref[s] = _gelu(jnp.concatenate([ht, hb], axis=1) + b1_ref[:])

    @pl.when(s == 1)
    def _():
        out_ref[:] = jnp.dot(hid_ref[0].astype(jnp.bfloat16),
                             c2_ref[:].astype(jnp.bfloat16).T,
                             preferred_element_type=jnp.float32)

    @pl.when(s == 2)
    def _():
        out_ref[:] = out_ref[:] + jnp.dot(hid_ref[1].astype(jnp.bfloat16),
                                          c2_ref[:].astype(jnp.bfloat16).T,
                                          preferred_element_type=jnp.float32) \
            + b2_ref[:]


def kernel(x, expert_emb, exp_fc1_w, exp_fc1_b, exp_fc2_w, exp_fc2_b,
           sw_fc1_w, sw_fc1_b, sw_fc2_w, sw_fc2_b,
           ch_fc1_w, ch_fc1_b, ch_fc2_w, ch_fc2_b):
    import functools
    bsz, ntok, dim = x.shape
    nexp = expert_emb.shape[0]
    ed = exp_fc1_w.shape[1]
    ncls = ch_fc2_w.shape[0]
    f32 = jnp.float32

    x_tok = x.reshape(bsz * ntok, dim)
    x_flat = x.reshape(bsz, ntok * dim)

    sel = pl.pallas_call(
        functools.partial(_router_kernel, bsz=bsz, ntok=ntok, dim=dim,
                          nexp=nexp),
        out_shape=jax.ShapeDtypeStruct((nexp, bsz, ed), f32),
    )(x_tok, expert_emb)

    SWB = 256
    nsteps = (ntok * dim) // (2 * SWB)
    b1_2d = sw_fc1_b.reshape(1, -1)

    def _w1spec(j):
        return pl.BlockSpec((SWB, ntok * dim), lambda s: (s + j * 16, 0))

    def _b1spec(j):
        return pl.BlockSpec((1, SWB), lambda s: (0, s + j * 16))

    def _w2spec(j):
        return pl.BlockSpec((nexp, SWB), lambda s: (0, s + j * 16))

    wts = pl.pallas_call(
        _sw_kernel,
        grid=(nsteps,),
        in_specs=[
            pl.BlockSpec((bsz, ntok * dim), lambda s: (0, 0)),
            _w1spec(0), _w1spec(1),
            _b1spec(0), _b1spec(1),
            _w2spec(0), _w2spec(1),
            pl.BlockSpec((1, nexp), lambda s: (0, 0)),
        ],
        out_specs=pl.BlockSpec((bsz, nexp), lambda s: (0, 0)),
        out_shape=jax.ShapeDtypeStruct((bsz, nexp), f32),
        scratch_shapes=[pltpu.VMEM((bsz, nexp), f32)],
        compiler_params=pltpu.CompilerParams(
            vmem_limit_bytes=60 * 1024 * 1024),
    )(x_flat, sw_fc1_w, sw_fc1_w, b1_2d, b1_2d,
      sw_fc2_w, sw_fc2_w, sw_fc2_b.reshape(1, -1))

    hed = ed // 2
    h1 = pl.pallas_call(
        _fc1_kernel,
        grid=(nexp,),
        in_specs=[
            pl.BlockSpec((1, bsz, ed), lambda e: (e, 0, 0)),
            pl.BlockSpec((1, hed, ed), lambda e: (e, 0, 0)),
            pl.BlockSpec((1, hed, ed), lambda e: (e, 1, 0)),
            pl.BlockSpec((1, 1, ed), lambda e: (e, 0, 0)),
        ],
        out_specs=pl.BlockSpec((1, bsz, ed), lambda e: (e, 0, 0)),
        out_shape=jax.ShapeDtypeStruct((nexp, bsz, ed), f32),
        compiler_params=pltpu.CompilerParams(
            vmem_limit_bytes=60 * 1024 * 1024),
    )(sel, exp_fc1_w, exp_fc1_w, exp_fc1_b.reshape(nexp, 1, ed))

    ws = pl.pallas_call(
        functools.partial(_fc2_kernel, nexp=nexp),
        grid=(nexp,),
        in_specs=[
            pl.BlockSpec((1, bsz, ed), lambda e: (e, 0, 0)),
            pl.BlockSpec((1, hed, ed), lambda e: (e, 0, 0)),
            pl.BlockSpec((1, hed, ed), lambda e: (e, 1, 0)),
            pl.BlockSpec((1, 1, ed), lambda e: (e, 0, 0)),
            pl.BlockSpec((bsz, nexp), lambda e: (0, 0)),
        ],
        out_specs=pl.BlockSpec((bsz, ed), lambda e: (0, 0)),
        out_shape=jax.ShapeDtypeStruct((bsz, ed), f32),
        compiler_params=pltpu.CompilerParams(
            vmem_limit_bytes=60 * 1024 * 1024),
    )(h1, exp_fc2_w, exp_fc2_w, exp_fc2_b.reshape(nexp, 1, ed), wts)

    qed = ed // 4
    out = pl.pallas_call(
        _head_kernel,
        grid=(3,),
        in_specs=[
            pl.BlockSpec((bsz, ed), lambda s: (0, 0)),
            pl.BlockSpec((qed, ed), lambda s: (2 * jnp.minimum(s, 1), 0)),
            pl.BlockSpec((qed, ed), lambda s: (2 * jnp.minimum(s, 1) + 1, 0)),
            pl.BlockSpec((1, ed // 2), lambda s: (0, jnp.minimum(s, 1))),
            pl.BlockSpec((ncls, ed // 2), lambda s: (0, jnp.clip(s - 1, 0, 1))),
            pl.BlockSpec((1, ncls), lambda s: (0, 0)),
        ],
        out_specs=pl.BlockSpec((bsz, ncls), lambda s: (0, 0)),
        out_shape=jax.ShapeDtypeStruct((bsz, ncls), f32),
        scratch_shapes=[pltpu.VMEM((2, bsz, ed // 2), f32)],
        compiler_params=pltpu.CompilerParams(
            vmem_limit_bytes=60 * 1024 * 1024),
    )(ws, ch_fc1_w, ch_fc1_w, ch_fc1_b.reshape(1, -1), ch_fc2_w,
      ch_fc2_b.reshape(1, -1))
    return out


# speedup vs baseline: 1.0595x; 1.0081x over previous
import jax
import jax.numpy as jnp
from jax.experimental import pallas as pl
from jax.experimental.pallas import tpu as pltpu

_HI = jax.lax.Precision.HIGHEST


def _gelu(v):
    return 0.5 * v * (1.0 + jax.lax.erf(v * 0.7071067811865475))


def _router_body(x_ref, emb_ref, sel_ref, *, bsz, ntok, dim, nexp):
    T = bsz * ntok
    x2 = x_ref[:]
    logits = jnp.dot(x2.astype(jnp.bfloat16), emb_ref[:].astype(jnp.bfloat16).T,
                     preferred_element_type=jnp.float32)
    col = jax.lax.broadcasted_iota(jnp.int32, (T, nexp), 1)
    m1 = jnp.max(logits, axis=1, keepdims=True)
    i1 = jnp.min(jnp.where(logits == m1, col, nexp), axis=1, keepdims=True)
    masked = jnp.where(col == i1, -jnp.inf, logits)
    m2 = jnp.max(masked, axis=1, keepdims=True)
    i2 = jnp.min(jnp.where(masked == m2, col, nexp), axis=1, keepdims=True)
    t = jax.lax.broadcasted_iota(jnp.int32, (T, 1), 0)
    base = t - t % ntok
    src = jnp.concatenate([(base + i1).astype(jnp.float32),
                           (base + i2).astype(jnp.float32)], axis=1)
    q = (t % bsz) * ntok + t // bsz
    colT = jax.lax.broadcasted_iota(jnp.int32, (T, T), 1)
    perm = (colT == q).astype(jnp.float32)
    srcp = jnp.dot(perm, src, preferred_element_type=jnp.float32,
                   precision=_HI)
    s1 = srcp[:, 0:1].astype(jnp.int32)
    s2 = srcp[:, 1:2].astype(jnp.int32)
    oh1 = (colT == s1).astype(jnp.float32)
    oh2 = (colT == s2).astype(jnp.float32)
    g1 = jnp.dot(oh1, x2, preferred_element_type=jnp.float32, precision=_HI)
    g2 = jnp.dot(oh2, x2, preferred_element_type=jnp.float32, precision=_HI)
    sel = jnp.concatenate([g1, g2], axis=1)
    sel_ref[:] = sel.reshape(nexp, bsz, 2 * dim)


def _sw_kernel(x_ref, xtok_ref, emb_ref, w1a_ref, w1b_ref, b1a_ref, b1b_ref,
               w2a_ref, w2b_ref, b2_ref, wts_ref, sel_ref,
               acc_ref, *, bsz, ntok, dim, nexp):
    s = pl.program_id(0)

    @pl.when(s == 0)
    def _():
        _router_body(xtok_ref, emb_ref, sel_ref,
                     bsz=bsz, ntok=ntok, dim=dim, nexp=nexp)

    xb = x_ref[:].astype(jnp.bfloat16)
    contrib = 0.0
    for w1_ref, b1_ref, w2_ref in ((w1a_ref, b1a_ref, w2a_ref),
                                   (w1b_ref, b1b_ref, w2b_ref)):
        h = _gelu(jnp.dot(xb, w1_ref[:].astype(jnp.bfloat16).T,
                          preferred_element_type=jnp.float32) + b1_ref[:])
        contrib = contrib + jnp.dot(
            h.astype(jnp.bfloat16), w2_ref[:].astype(jnp.bfloat16).T,
            preferred_element_type=jnp.float32)

    @pl.when(s == 0)
    def _():
        acc_ref[:] = contrib

    @pl.when(s > 0)
    def _():
        acc_ref[:] = acc_ref[:] + contrib

    @pl.when(s == pl.num_programs(0) - 1)
    def _():
        logits = acc_ref[:] + b2_ref[:]
        m = jnp.max(logits, axis=1, keepdims=True)
        ez = jnp.exp(logits - m)
        wts_ref[:] = ez / jnp.sum(ez, axis=1, keepdims=True)


def _fc1_kernel(sel_ref, wt_ref, wb_ref, b_ref, h_ref):
    sb = sel_ref[0].astype(jnp.bfloat16)
    ht = jnp.dot(sb, wt_ref[0].astype(jnp.bfloat16).T,
                 preferred_element_type=jnp.float32)
    hb = jnp.dot(sb, wb_ref[0].astype(jnp.bfloat16).T,
                 preferred_element_type=jnp.float32)
    h = jnp.concatenate([ht, hb], axis=1) + b_ref[0]
    h_ref[0] = _gelu(h)


def _fc2_kernel(h_ref, wt_ref, wb_ref, b_ref, wts_ref, out_ref, *, nexp):
    e = pl.program_id(0)
    hb16 = h_ref[0].astype(jnp.bfloat16)
    rt = jnp.dot(hb16, wt_ref[0].astype(jnp.bfloat16).T,
                 preferred_element_type=jnp.float32)
    rb = jnp.dot(hb16, wb_ref[0].astype(jnp.bfloat16).T,
                 preferred_element_type=jnp.float32)
    r = jnp.concatenate([rt, rb], axis=1) + b_ref[0]
    ecol = jax.lax.broadcasted_iota(jnp.int32, (nexp, 1), 0)
    onehot = (ecol == e).astype(jnp.float32)
    wcol = jnp.dot(wts_ref[:], onehot, preferred_element_type=jnp.float32,
                   precision=_HI)
    contrib = r * wcol

    @pl.when(e == 0)
    def _():
        out_ref[:] = contrib

    @pl.when(e > 0)
    def _():
        out_ref[:] = out_ref[:] + contrib


def _head_kernel(ws_ref, c1a_ref, c1b_ref, b1_ref, c2_ref, b2_ref, out_ref,
                 hid_ref):
    s = pl.program_id(0)

    @pl.when(s < 2)
    def _():
        wsb = ws_ref[:].astype(jnp.bfloat16)
        ht = jnp.dot(wsb, c1a_ref[:].astype(jnp.bfloat16).T,
                     preferred_element_type=jnp.float32)
        hb = jnp.dot(wsb, c1b_ref[:].astype(jnp.bfloat16).T,
                     preferred_element_type=jnp.float32)
        hid_ref[s] = _gelu(jnp.concatenate([ht, hb], axis=1) + b1_ref[:])

    @pl.when(s == 1)
    def _():
        out_ref[:] = jnp.dot(hid_ref[0].astype(jnp.bfloat16),
                             c2_ref[:].astype(jnp.bfloat16).T,
                             preferred_element_type=jnp.float32)

    @pl.when(s == 2)
    def _():
        out_ref[:] = out_ref[:] + jnp.dot(hid_ref[1].astype(jnp.bfloat16),
                                          c2_ref[:].astype(jnp.bfloat16).T,
                                          preferred_element_type=jnp.float32) \
            + b2_ref[:]


def kernel(x, expert_emb, exp_fc1_w, exp_fc1_b, exp_fc2_w, exp_fc2_b,
           sw_fc1_w, sw_fc1_b, sw_fc2_w, sw_fc2_b,
           ch_fc1_w, ch_fc1_b, ch_fc2_w, ch_fc2_b):
    import functools
    bsz, ntok, dim = x.shape
    nexp = expert_emb.shape[0]
    ed = exp_fc1_w.shape[1]
    ncls = ch_fc2_w.shape[0]
    f32 = jnp.float32

    x_tok = x.reshape(bsz * ntok, dim)
    x_flat = x.reshape(bsz, ntok * dim)

    SWB = 256
    nsteps = (ntok * dim) // (2 * SWB)
    b1_2d = sw_fc1_b.reshape(1, -1)

    def _w1spec(j):
        return pl.BlockSpec((SWB, ntok * dim), lambda s: (s + j * 16, 0))

    def _b1spec(j):
        return pl.BlockSpec((1, SWB), lambda s: (0, s + j * 16))

    def _w2spec(j):
        return pl.BlockSpec((nexp, SWB), lambda s: (0, s + j * 16))

    wts, sel = pl.pallas_call(
        functools.partial(_sw_kernel, bsz=bsz, ntok=ntok, dim=dim, nexp=nexp),
        grid=(nsteps,),
        in_specs=[
            pl.BlockSpec((bsz, ntok * dim), lambda s: (0, 0)),
            pl.BlockSpec((bsz * ntok, dim), lambda s: (0, 0)),
            pl.BlockSpec((nexp, dim), lambda s: (0, 0)),
            _w1spec(0), _w1spec(1),
            _b1spec(0), _b1spec(1),
            _w2spec(0), _w2spec(1),
            pl.BlockSpec((1, nexp), lambda s: (0, 0)),
        ],
        out_specs=[pl.BlockSpec((bsz, nexp), lambda s: (0, 0)),
                   pl.BlockSpec((nexp, bsz, ed), lambda s: (0, 0, 0))],
        out_shape=[jax.ShapeDtypeStruct((bsz, nexp), f32),
                   jax.ShapeDtypeStruct((nexp, bsz, ed), f32)],
        scratch_shapes=[pltpu.VMEM((bsz, nexp), f32)],
        compiler_params=pltpu.CompilerParams(
            vmem_limit_bytes=60 * 1024 * 1024),
    )(x_flat, x_tok, expert_emb, sw_fc1_w, sw_fc1_w, b1_2d, b1_2d,
      sw_fc2_w, sw_fc2_w, sw_fc2_b.reshape(1, -1))

    hed = ed // 2
    h1 = pl.pallas_call(
        _fc1_kernel,
        grid=(nexp,),
        in_specs=[
            pl.BlockSpec((1, bsz, ed), lambda e: (e, 0, 0)),
            pl.BlockSpec((1, hed, ed), lambda e: (e, 0, 0)),
            pl.BlockSpec((1, hed, ed), lambda e: (e, 1, 0)),
            pl.BlockSpec((1, 1, ed), lambda e: (e, 0, 0)),
        ],
        out_specs=pl.BlockSpec((1, bsz, ed), lambda e: (e, 0, 0)),
        out_shape=jax.ShapeDtypeStruct((nexp, bsz, ed), f32),
        compiler_params=pltpu.CompilerParams(
            vmem_limit_bytes=60 * 1024 * 1024),
    )(sel, exp_fc1_w, exp_fc1_w, exp_fc1_b.reshape(nexp, 1, ed))

    ws = pl.pallas_call(
        functools.partial(_fc2_kernel, nexp=nexp),
        grid=(nexp,),
        in_specs=[
            pl.BlockSpec((1, bsz, ed), lambda e: (e, 0, 0)),
            pl.BlockSpec((1, hed, ed), lambda e: (e, 0, 0)),
            pl.BlockSpec((1, hed, ed), lambda e: (e, 1, 0)),
            pl.BlockSpec((1, 1, ed), lambda e: (e, 0, 0)),
            pl.BlockSpec((bsz, nexp), lambda e: (0, 0)),
        ],
        out_specs=pl.BlockSpec((bsz, ed), lambda e: (0, 0)),
        out_shape=jax.ShapeDtypeStruct((bsz, ed), f32),
        compiler_params=pltpu.CompilerParams(
            vmem_limit_bytes=60 * 1024 * 1024),
    )(h1, exp_fc2_w, exp_fc2_w, exp_fc2_b.reshape(nexp, 1, ed), wts)

    qed = ed // 4
    out = pl.pallas_call(
        _head_kernel,
        grid=(3,),
        in_specs=[
            pl.BlockSpec((bsz, ed), lambda s: (0, 0)),
            pl.BlockSpec((qed, ed), lambda s: (2 * jnp.minimum(s, 1), 0)),
            pl.BlockSpec((qed, ed), lambda s: (2 * jnp.minimum(s, 1) + 1, 0)),
            pl.BlockSpec((1, ed // 2), lambda s: (0, jnp.minimum(s, 1))),
            pl.BlockSpec((ncls, ed // 2), lambda s: (0, jnp.clip(s - 1, 0, 1))),
            pl.BlockSpec((1, ncls), lambda s: (0, 0)),
        ],
        out_specs=pl.BlockSpec((bsz, ncls), lambda s: (0, 0)),
        out_shape=jax.ShapeDtypeStruct((bsz, ncls), f32),
        scratch_shapes=[pltpu.VMEM((2, bsz, ed // 2), f32)],
        compiler_params=pltpu.CompilerParams(
            vmem_limit_bytes=60 * 1024 * 1024),
    )(ws, ch_fc1_w, ch_fc1_w, ch_fc1_b.reshape(1, -1), ch_fc2_w,
      ch_fc2_b.reshape(1, -1))
    return out


# speedup vs baseline: 1.0683x; 1.0083x over previous
import jax
import jax.numpy as jnp
from jax.experimental import pallas as pl
from jax.experimental.pallas import tpu as pltpu

_HI = jax.lax.Precision.HIGHEST


def _gelu(v):
    return 0.5 * v * (1.0 + jax.lax.erf(v * 0.7071067811865475))


def _router_body(x_ref, emb_ref, sel_ref, *, bsz, ntok, dim, nexp):
    T = bsz * ntok
    x2 = x_ref[:]
    logits = jnp.dot(x2.astype(jnp.bfloat16), emb_ref[:].astype(jnp.bfloat16).T,
                     preferred_element_type=jnp.float32)
    col = jax.lax.broadcasted_iota(jnp.int32, (T, nexp), 1)
    m1 = jnp.max(logits, axis=1, keepdims=True)
    i1 = jnp.min(jnp.where(logits == m1, col, nexp), axis=1, keepdims=True)
    masked = jnp.where(col == i1, -jnp.inf, logits)
    m2 = jnp.max(masked, axis=1, keepdims=True)
    i2 = jnp.min(jnp.where(masked == m2, col, nexp), axis=1, keepdims=True)
    t = jax.lax.broadcasted_iota(jnp.int32, (T, 1), 0)
    base = t - t % ntok
    src = jnp.concatenate([(base + i1).astype(jnp.float32),
                           (base + i2).astype(jnp.float32)], axis=1)
    q = (t % bsz) * ntok + t // bsz
    colT = jax.lax.broadcasted_iota(jnp.int32, (T, T), 1)
    perm = (colT == q).astype(jnp.float32)
    srcp = jnp.dot(perm, src, preferred_element_type=jnp.float32,
                   precision=_HI)
    s1 = srcp[:, 0:1].astype(jnp.int32)
    s2 = srcp[:, 1:2].astype(jnp.int32)
    oh1 = (colT == s1).astype(jnp.float32)
    oh2 = (colT == s2).astype(jnp.float32)
    g1 = jnp.dot(oh1, x2, preferred_element_type=jnp.float32, precision=_HI)
    g2 = jnp.dot(oh2, x2, preferred_element_type=jnp.float32, precision=_HI)
    sel = jnp.concatenate([g1, g2], axis=1)
    sel_ref[:] = sel.reshape(nexp, bsz, 2 * dim)


def _sw_kernel(x_ref, xtok_ref, emb_ref, w1a_ref, w1b_ref, b1a_ref, b1b_ref,
               w2a_ref, w2b_ref, b2_ref, wts_ref, sel_ref,
               acc_ref, *, bsz, ntok, dim, nexp):
    s = pl.program_id(0)

    @pl.when(s == 0)
    def _():
        _router_body(xtok_ref, emb_ref, sel_ref,
                     bsz=bsz, ntok=ntok, dim=dim, nexp=nexp)

    xb = x_ref[:].astype(jnp.bfloat16)
    contrib = 0.0
    for w1_ref, b1_ref, w2_ref in ((w1a_ref, b1a_ref, w2a_ref),
                                   (w1b_ref, b1b_ref, w2b_ref)):
        h = _gelu(jnp.dot(xb, w1_ref[:].astype(jnp.bfloat16).T,
                          preferred_element_type=jnp.float32) + b1_ref[:])
        contrib = contrib + jnp.dot(
            h.astype(jnp.bfloat16), w2_ref[:].astype(jnp.bfloat16).T,
            preferred_element_type=jnp.float32)

    @pl.when(s == 0)
    def _():
        acc_ref[:] = contrib

    @pl.when(s > 0)
    def _():
        acc_ref[:] = acc_ref[:] + contrib

    @pl.when(s == pl.num_programs(0) - 1)
    def _():
        logits = acc_ref[:] + b2_ref[:]
        m = jnp.max(logits, axis=1, keepdims=True)
        ez = jnp.exp(logits - m)
        wts_ref[:] = ez / jnp.sum(ez, axis=1, keepdims=True)


def _experts_kernel(sel_ref, f1a_ref, f1b_ref, b1_ref,
                    f2a_ref, f2b_ref, b2_ref, wts_ref, ws_ref,
                    h1_ref, *, nexp, nfc1):
    s = pl.program_id(0)
    hd = h1_ref.shape[3]

    @pl.when(s < nfc1)
    def _fc1():
        e = s // 2
        h = s % 2
        sb = sel_ref[e].astype(jnp.bfloat16)
        ht = jnp.dot(sb, f1a_ref[0].astype(jnp.bfloat16).T,
                     preferred_element_type=jnp.float32)
        hb = jnp.dot(sb, f1b_ref[0].astype(jnp.bfloat16).T,
                     preferred_element_type=jnp.float32)
        h1_ref[e, h] = _gelu(jnp.concatenate([ht, hb], axis=1) + b1_ref[0, 0])

    @pl.when(s >= nfc1)
    def _fc2():
        t = s - nfc1
        e = t // 2
        o = t % 2
        h1a = h1_ref[e, 0].astype(jnp.bfloat16)
        h1b = h1_ref[e, 1].astype(jnp.bfloat16)
        wa = f2a_ref[0].astype(jnp.bfloat16)
        wb = f2b_ref[0].astype(jnp.bfloat16)
        rt = (jnp.dot(h1a, wa[:, 0:hd].T, preferred_element_type=jnp.float32)
              + jnp.dot(h1b, wa[:, hd:2 * hd].T,
                        preferred_element_type=jnp.float32))
        rb = (jnp.dot(h1a, wb[:, 0:hd].T, preferred_element_type=jnp.float32)
              + jnp.dot(h1b, wb[:, hd:2 * hd].T,
                        preferred_element_type=jnp.float32))
        r = jnp.concatenate([rt, rb], axis=1) + b2_ref[0, 0]
        ecol = jax.lax.broadcasted_iota(jnp.int32, (nexp, 1), 0)
        wcol = jnp.dot(wts_ref[:], (ecol == e).astype(jnp.float32),
                       preferred_element_type=jnp.float32,
                       precision=_HI)
        contrib = r * wcol

        @pl.when(e == 0)
        def _():
            ws_ref[o] = contrib

        @pl.when(e > 0)
        def _():
            ws_ref[o] = ws_ref[o] + contrib


def _head_kernel(ws_ref, c1a_ref, c1b_ref, b1_ref, c2_ref, b2_ref, out_ref,
                 hid_ref):
    s = pl.program_id(0)

    @pl.when(s < 2)
    def _():
        hd = ws_ref.shape[2]
        ws0 = ws_ref[0].astype(jnp.bfloat16)
        ws1 = ws_ref[1].astype(jnp.bfloat16)
        c1a = c1a_ref[:].astype(jnp.bfloat16)
        c1b = c1b_ref[:].astype(jnp.bfloat16)
        ht = (jnp.dot(ws0, c1a[:, 0:hd].T, preferred_element_type=jnp.float32)
              + jnp.dot(ws1, c1a[:, hd:2 * hd].T,
                        preferred_element_type=jnp.float32))
        hb = (jnp.dot(ws0, c1b[:, 0:hd].T, preferred_element_type=jnp.float32)
              + jnp.dot(ws1, c1b[:, hd:2 * hd].T,
                        preferred_element_type=jnp.float32))
        hid_ref[s] = _gelu(jnp.concatenate([ht, hb], axis=1) + b1_ref[:])

    @pl.when(s == 1)
    def _():
        out_ref[:] = jnp.dot(hid_ref[0].astype(jnp.bfloat16),
                             c2_ref[:].astype(jnp.bfloat16).T,
                             preferred_element_type=jnp.float32)

    @pl.when(s == 2)
    def _():
        out_ref[:] = out_ref[:] + jnp.dot(hid_ref[1].astype(jnp.bfloat16),
                                          c2_ref[:].astype(jnp.bfloat16).T,
                                          preferred_element_type=jnp.float32) \
            + b2_ref[:]


def kernel(x, expert_emb, exp_fc1_w, exp_fc1_b, exp_fc2_w, exp_fc2_b,
           sw_fc1_w, sw_fc1_b, sw_fc2_w, sw_fc2_b,
           ch_fc1_w, ch_fc1_b, ch_fc2_w, ch_fc2_b):
    import functools
    bsz, ntok, dim = x.shape
    nexp = expert_emb.shape[0]
    ed = exp_fc1_w.shape[1]
    ncls = ch_fc2_w.shape[0]
    f32 = jnp.float32

    x_tok = x.reshape(bsz * ntok, dim)
    x_flat = x.reshape(bsz, ntok * dim)

    SWB = 256
    nsteps = (ntok * dim) // (2 * SWB)
    b1_2d = sw_fc1_b.reshape(1, -1)

    def _w1spec(j):
        return pl.BlockSpec((SWB, ntok * dim), lambda s: (s + j * 16, 0))

    def _b1spec(j):
        return pl.BlockSpec((1, SWB), lambda s: (0, s + j * 16))

    def _w2spec(j):
        return pl.BlockSpec((nexp, SWB), lambda s: (0, s + j * 16))

    wts, sel = pl.pallas_call(
        functools.partial(_sw_kernel, bsz=bsz, ntok=ntok, dim=dim, nexp=nexp),
        grid=(nsteps,),
        in_specs=[
            pl.BlockSpec((bsz, ntok * dim), lambda s: (0, 0)),
            pl.BlockSpec((bsz * ntok, dim), lambda s: (0, 0)),
            pl.BlockSpec((nexp, dim), lambda s: (0, 0)),
            _w1spec(0), _w1spec(1),
            _b1spec(0), _b1spec(1),
            _w2spec(0), _w2spec(1),
            pl.BlockSpec((1, nexp), lambda s: (0, 0)),
        ],
        out_specs=[pl.BlockSpec((bsz, nexp), lambda s: (0, 0)),
                   pl.BlockSpec((nexp, bsz, ed), lambda s: (0, 0, 0))],
        out_shape=[jax.ShapeDtypeStruct((bsz, nexp), f32),
                   jax.ShapeDtypeStruct((nexp, bsz, ed), f32)],
        scratch_shapes=[pltpu.VMEM((bsz, nexp), f32)],
        compiler_params=pltpu.CompilerParams(
            vmem_limit_bytes=60 * 1024 * 1024),
    )(x_flat, x_tok, expert_emb, sw_fc1_w, sw_fc1_w, b1_2d, b1_2d,
      sw_fc2_w, sw_fc2_w, sw_fc2_b.reshape(1, -1))

    qed = ed // 4
    hd = ed // 2
    nfc1 = 2 * nexp

    def _f1a(s):
        t = jnp.minimum(s, nfc1 - 1)
        return t // 2, 2 * (t % 2), 0

    def _f1b(s):
        t = jnp.minimum(s, nfc1 - 1)
        return t // 2, 2 * (t % 2) + 1, 0

    def _f2a(s):
        t = jnp.clip(s - nfc1, 0, nfc1 - 1)
        return t // 2, 2 * (t % 2), 0

    def _f2b(s):
        t = jnp.clip(s - nfc1, 0, nfc1 - 1)
        return t // 2, 2 * (t % 2) + 1, 0

    ws = pl.pallas_call(
        functools.partial(_experts_kernel, nexp=nexp, nfc1=nfc1),
        grid=(2 * nfc1,),
        in_specs=[
            pl.BlockSpec((nexp, bsz, ed), lambda s: (0, 0, 0)),
            pl.BlockSpec((1, qed, ed), _f1a),
            pl.BlockSpec((1, qed, ed), _f1b),
            pl.BlockSpec((1, 1, 1, hd),
                         lambda s: (jnp.minimum(s, nfc1 - 1) // 2,
                                    jnp.minimum(s, nfc1 - 1) % 2, 0, 0)),
            pl.BlockSpec((1, qed, ed), _f2a),
            pl.BlockSpec((1, qed, ed), _f2b),
            pl.BlockSpec((1, 1, 1, hd),
                         lambda s: (jnp.clip(s - nfc1, 0, nfc1 - 1) // 2,
                                    jnp.clip(s - nfc1, 0, nfc1 - 1) % 2, 0, 0)),
            pl.BlockSpec((bsz, nexp), lambda s: (0, 0)),
        ],
        out_specs=pl.BlockSpec((2, bsz, hd), lambda s: (0, 0, 0)),
        out_shape=jax.ShapeDtypeStruct((2, bsz, hd), f32),
        scratch_shapes=[pltpu.VMEM((nexp, 2, bsz, hd), f32)],
        compiler_params=pltpu.CompilerParams(
            vmem_limit_bytes=60 * 1024 * 1024),
    )(sel, exp_fc1_w, exp_fc1_w, exp_fc1_b.reshape(nexp, 2, 1, hd),
      exp_fc2_w, exp_fc2_w, exp_fc2_b.reshape(nexp, 2, 1, hd), wts)

    out = pl.pallas_call(
        _head_kernel,
        grid=(3,),
        in_specs=[
            pl.BlockSpec((2, bsz, hd), lambda s: (0, 0, 0)),
            pl.BlockSpec((qed, ed), lambda s: (2 * jnp.minimum(s, 1), 0)),
            pl.BlockSpec((qed, ed), lambda s: (2 * jnp.minimum(s, 1) + 1, 0)),
            pl.BlockSpec((1, ed // 2), lambda s: (0, jnp.minimum(s, 1))),
            pl.BlockSpec((ncls, ed // 2), lambda s: (0, jnp.clip(s - 1, 0, 1))),
            pl.BlockSpec((1, ncls), lambda s: (0, 0)),
        ],
        out_specs=pl.BlockSpec((bsz, ncls), lambda s: (0, 0)),
        out_shape=jax.ShapeDtypeStruct((bsz, ncls), f32),
        scratch_shapes=[pltpu.VMEM((2, bsz, ed // 2), f32)],
        compiler_params=pltpu.CompilerParams(
            vmem_limit_bytes=60 * 1024 * 1024),
    )(ws, ch_fc1_w, ch_fc1_w, ch_fc1_b.reshape(1, -1), ch_fc2_w,
      ch_fc2_b.reshape(1, -1))
    return out
